# baseline (speedup 1.0000x reference)
import functools

import jax
import jax.numpy as jnp
from jax import lax
from jax.experimental import pallas as pl
from jax.experimental.pallas import tpu as pltpu
from jax.experimental.pallas import tpu_sc as plsc

NUM_CHAMPS = 164
NUM_ROLES = 6
CHAMP_DIM = 32
PAIRS = NUM_CHAMPS * NUM_ROLES
OUT = 10
NC, NS = 2, 16
NW = NC * NS

ROWS = 8
COLS = 256


def _pair_table_body(ctT, rtT, w1T, b1r, w2T, b2r, out):
    w1 = w1T[...]
    cpT = jnp.dot(w1[:, :CHAMP_DIM], ctT[...],
                  preferred_element_type=jnp.float32)
    rpT = jnp.dot(w1[:, CHAMP_DIM:], rtT[...],
                  preferred_element_type=jnp.float32)
    pc = lax.broadcasted_iota(jnp.int32, (NUM_CHAMPS, PAIRS), 1) // NUM_ROLES
    ec = (pc == lax.broadcasted_iota(jnp.int32, (NUM_CHAMPS, PAIRS), 0)).astype(jnp.float32)
    pr = lax.broadcasted_iota(jnp.int32, (NUM_ROLES, PAIRS), 1) % NUM_ROLES
    er = (pr == lax.broadcasted_iota(jnp.int32, (NUM_ROLES, PAIRS), 0)).astype(jnp.float32)
    hT = jnp.maximum(
        jnp.dot(cpT, ec, preferred_element_type=jnp.float32)
        + jnp.dot(rpT, er, preferred_element_type=jnp.float32)
        + b1r[...].T,
        0.0,
    )
    out[...] = (jnp.dot(w2T[...], hT, preferred_element_type=jnp.float32)
                + b2r[...].T)


def _build_pair_table(champ_table, role_table, W1, b1, W2, b2):
    ctT = jnp.swapaxes(champ_table, 0, 1)
    rtT = jnp.swapaxes(role_table, 0, 1)
    W1T = jnp.swapaxes(W1, 0, 1)
    w2T = jnp.swapaxes(W2, 0, 1)
    return pl.pallas_call(
        _pair_table_body,
        out_shape=jax.ShapeDtypeStruct((OUT, PAIRS), jnp.float32),
    )(ctT, rtT, W1T, b1.reshape(1, -1), w2T, b2.reshape(1, -1))


def _gather_body(chunks_per_w, col_blocks, table_hbm, champ_hbm, role_hbm,
                 out_hbm, table_v, cbuf, rbuf, obuf, csem, rsem, osem):
    wid = lax.axis_index("s") * NC + lax.axis_index("c")
    pltpu.sync_copy(table_hbm, table_v)

    def splat(x):
        return jnp.full((16,), x, jnp.int32)

    def offsets(i):
        t = wid * chunks_per_w + i
        lb = t // col_blocks
        bb = t - lb * col_blocks
        return lb * ROWS, bb * COLS

    def in_copies(i, slot):
        l0, b0 = offsets(i)
        src = lambda ref: ref.at[pl.ds(l0, ROWS), pl.ds(b0, COLS)]
        return (
            pltpu.make_async_copy(src(champ_hbm), cbuf.at[slot], csem.at[slot]),
            pltpu.make_async_copy(src(role_hbm), rbuf.at[slot], rsem.at[slot]),
        )

    def out_copy(i, slot):
        l0, b0 = offsets(i)
        return pltpu.make_async_copy(
            obuf.at[slot], out_hbm.at[:, pl.ds(l0, ROWS), pl.ds(b0, COLS)],
            osem.at[slot])

    for cp in in_copies(0, 0):
        cp.start()

    def chunk_body(i, _):
        slot = lax.rem(i, 2)
        nxt = 1 - slot

        @pl.when(i + 1 < chunks_per_w)
        def _():
            for cp in in_copies(i + 1, nxt):
                cp.start()

        for cp in in_copies(i, slot):
            cp.wait()

        @pl.when(i >= 2)
        def _():
            out_copy(i - 2, slot).wait()

        def row_body(r, _):
            @plsc.parallel_loop(0, COLS // 16, unroll=8)
            def grp_body(g):
                c = cbuf[slot, r, pl.ds(g * 16, 16)]
                rr = rbuf[slot, r, pl.ds(g * 16, 16)]
                idx = c * splat(NUM_ROLES) + rr
                for d in range(OUT):
                    v = plsc.load_gather(table_v, [idx + splat(d * PAIRS)])
                    obuf[slot, d, r, pl.ds(g * 16, 16)] = v

            return 0

        lax.fori_loop(0, ROWS, row_body, 0)
        out_copy(i, slot).start()
        return 0

    lax.fori_loop(0, chunks_per_w, chunk_body, 0)
    out_copy(chunks_per_w - 2, lax.rem(chunks_per_w - 2, 2)).wait()
    out_copy(chunks_per_w - 1, lax.rem(chunks_per_w - 1, 2)).wait()


@functools.cache
def _make_gather(batch, seq_len):
    n_chunks = (seq_len // ROWS) * (batch // COLS)
    assert n_chunks % NW == 0
    chunks_per_w = n_chunks // NW
    col_blocks = batch // COLS
    mesh = plsc.VectorSubcoreMesh(core_axis_name="c", subcore_axis_name="s")
    return pl.kernel(
        functools.partial(_gather_body, chunks_per_w, col_blocks),
        out_type=jax.ShapeDtypeStruct((OUT, seq_len, batch), jnp.float32),
        mesh=mesh,
        compiler_params=pltpu.CompilerParams(needs_layout_passes=False),
        scratch_types=[
            pltpu.VMEM((PAIRS * OUT,), jnp.float32),
            pltpu.VMEM((2, ROWS, COLS), jnp.int32),
            pltpu.VMEM((2, ROWS, COLS), jnp.int32),
            pltpu.VMEM((2, OUT, ROWS, COLS), jnp.float32),
            pltpu.SemaphoreType.DMA((2,)),
            pltpu.SemaphoreType.DMA((2,)),
            pltpu.SemaphoreType.DMA((2,)),
        ],
    )


def kernel(champions, roles, champ_table, role_table, W1, b1, W2, b2):
    B, L = champions.shape
    pair_table = _build_pair_table(champ_table, role_table, W1, b1, W2, b2)
    gather = _make_gather(B, L)
    out_t = gather(pair_table.reshape(-1), jnp.swapaxes(champions, 0, 1),
                   jnp.swapaxes(roles, 0, 1))
    return jnp.transpose(out_t, (2, 1, 0))


# speedup vs baseline: 1.2339x; 1.2339x over previous
import functools

import jax
import jax.numpy as jnp
from jax import lax
from jax.experimental import pallas as pl
from jax.experimental.pallas import tpu as pltpu
from jax.experimental.pallas import tpu_sc as plsc

NUM_CHAMPS = 164
NUM_ROLES = 6
CHAMP_DIM = 32
PAIRS = NUM_CHAMPS * NUM_ROLES
OUT = 10
NC, NS = 2, 16
NW = NC * NS

ROWS = 8
COLS = 512


def _pair_table_body(ctT, rtT, w1T, b1r, w2T, b2r, out):
    w1 = w1T[...]
    cpT = jnp.dot(w1[:, :CHAMP_DIM], ctT[...],
                  preferred_element_type=jnp.float32)
    rpT = jnp.dot(w1[:, CHAMP_DIM:], rtT[...],
                  preferred_element_type=jnp.float32)
    pc = lax.broadcasted_iota(jnp.int32, (NUM_CHAMPS, PAIRS), 1) // NUM_ROLES
    ec = (pc == lax.broadcasted_iota(jnp.int32, (NUM_CHAMPS, PAIRS), 0)).astype(jnp.float32)
    pr = lax.broadcasted_iota(jnp.int32, (NUM_ROLES, PAIRS), 1) % NUM_ROLES
    er = (pr == lax.broadcasted_iota(jnp.int32, (NUM_ROLES, PAIRS), 0)).astype(jnp.float32)
    hT = jnp.maximum(
        jnp.dot(cpT, ec, preferred_element_type=jnp.float32)
        + jnp.dot(rpT, er, preferred_element_type=jnp.float32)
        + b1r[...].T,
        0.0,
    )
    out[...] = (jnp.dot(w2T[...], hT, preferred_element_type=jnp.float32)
                + b2r[...].T)


def _build_pair_table(champ_table, role_table, W1, b1, W2, b2):
    ctT = jnp.swapaxes(champ_table, 0, 1)
    rtT = jnp.swapaxes(role_table, 0, 1)
    W1T = jnp.swapaxes(W1, 0, 1)
    w2T = jnp.swapaxes(W2, 0, 1)
    return pl.pallas_call(
        _pair_table_body,
        out_shape=jax.ShapeDtypeStruct((OUT, PAIRS), jnp.float32),
    )(ctT, rtT, W1T, b1.reshape(1, -1), w2T, b2.reshape(1, -1))


def _gather_body(chunks_per_w, col_blocks, table_hbm, champ_hbm, role_hbm,
                 out_hbm, table_v, cbuf, rbuf, obuf, csem, rsem, osem):
    wid = lax.axis_index("s") * NC + lax.axis_index("c")

    def splat(x):
        return jnp.full((16,), x, jnp.int32)

    def offsets(i):
        t = wid * chunks_per_w + i
        lb = t // col_blocks
        bb = t - lb * col_blocks
        return lb * ROWS, bb * COLS

    def in_copies(i, slot):
        l0, b0 = offsets(i)
        src = lambda ref: ref.at[pl.ds(l0, ROWS), pl.ds(b0, COLS)]
        return (
            pltpu.make_async_copy(src(champ_hbm), cbuf.at[slot], csem.at[slot]),
            pltpu.make_async_copy(src(role_hbm), rbuf.at[slot], rsem.at[slot]),
        )

    def out_copy(i, slot):
        l0, b0 = offsets(i)
        return pltpu.make_async_copy(
            obuf.at[slot], out_hbm.at[:, pl.ds(l0, ROWS), pl.ds(b0, COLS)],
            osem.at[slot])

    for cp in in_copies(0, 0):
        cp.start()
    pltpu.sync_copy(table_hbm, table_v)

    def chunk_body(i, _):
        slot = lax.rem(i, 2)
        nxt = 1 - slot

        @pl.when(i + 1 < chunks_per_w)
        def _():
            for cp in in_copies(i + 1, nxt):
                cp.start()

        for cp in in_copies(i, slot):
            cp.wait()

        @pl.when(i >= 2)
        def _():
            out_copy(i - 2, slot).wait()

        def row_body(r, _):
            @plsc.parallel_loop(0, COLS // 16, unroll=8)
            def grp_body(g):
                c = cbuf[slot, r, pl.ds(g * 16, 16)]
                rr = rbuf[slot, r, pl.ds(g * 16, 16)]
                idx = c * splat(NUM_ROLES) + rr
                for d in range(OUT):
                    v = plsc.load_gather(table_v, [idx + splat(d * PAIRS)])
                    obuf[slot, d, r, pl.ds(g * 16, 16)] = v

            return 0

        lax.fori_loop(0, ROWS, row_body, 0)
        out_copy(i, slot).start()
        return 0

    lax.fori_loop(0, chunks_per_w, chunk_body, 0)
    out_copy(chunks_per_w - 2, lax.rem(chunks_per_w - 2, 2)).wait()
    out_copy(chunks_per_w - 1, lax.rem(chunks_per_w - 1, 2)).wait()


@functools.cache
def _make_gather(batch, seq_len):
    n_chunks = (seq_len // ROWS) * (batch // COLS)
    assert n_chunks % NW == 0
    chunks_per_w = n_chunks // NW
    col_blocks = batch // COLS
    mesh = plsc.VectorSubcoreMesh(core_axis_name="c", subcore_axis_name="s")
    return pl.kernel(
        functools.partial(_gather_body, chunks_per_w, col_blocks),
        out_type=jax.ShapeDtypeStruct((OUT, seq_len, batch), jnp.float32),
        mesh=mesh,
        compiler_params=pltpu.CompilerParams(needs_layout_passes=False),
        scratch_types=[
            pltpu.VMEM((PAIRS * OUT,), jnp.float32),
            pltpu.VMEM((2, ROWS, COLS), jnp.int32),
            pltpu.VMEM((2, ROWS, COLS), jnp.int32),
            pltpu.VMEM((2, OUT, ROWS, COLS), jnp.float32),
            pltpu.SemaphoreType.DMA((2,)),
            pltpu.SemaphoreType.DMA((2,)),
            pltpu.SemaphoreType.DMA((2,)),
        ],
    )


def kernel(champions, roles, champ_table, role_table, W1, b1, W2, b2):
    B, L = champions.shape
    pair_table = _build_pair_table(champ_table, role_table, W1, b1, W2, b2)
    gather = _make_gather(B, L)
    out_t = gather(pair_table.reshape(-1), jnp.swapaxes(champions, 0, 1),
                   jnp.swapaxes(roles, 0, 1))
    return jnp.transpose(out_t, (2, 1, 0))
